# baseline (speedup 1.0000x reference)
import jax
import jax.numpy as jnp
from jax.experimental import pallas as pl
from jax.experimental.pallas import tpu as pltpu

N_USERS = 4096
X_DIM = 64
K_SUBS = 20000
R_DIM = 3

BK = 800
NK = K_SUBS // BK


def _body(st_ref, r_ref, o_ref, acc_ref):
    k = pl.program_id(0)

    @pl.when(k == 0)
    def _init():
        acc_ref[...] = jnp.zeros_like(acc_ref)

    acc_ref[...] += jax.lax.dot_general(
        st_ref[...],
        r_ref[...],
        (((0,), (0,)), ((), ())),
        preferred_element_type=jnp.float32,
    )

    @pl.when(k == NK - 1)
    def _fin():
        o_ref[...] = acc_ref[...]


def kernel(x, S, R):
    agg = pl.pallas_call(
        _body,
        grid=(NK,),
        in_specs=[
            pl.BlockSpec((BK, N_USERS), lambda k: (k, 0)),
            pl.BlockSpec((BK, R_DIM), lambda k: (k, 0)),
        ],
        out_specs=pl.BlockSpec((N_USERS, R_DIM), lambda k: (0, 0)),
        out_shape=jax.ShapeDtypeStruct((N_USERS, R_DIM), jnp.float32),
        scratch_shapes=[pltpu.VMEM((N_USERS, R_DIM), jnp.float32)],
        compiler_params=pltpu.CompilerParams(
            dimension_semantics=("arbitrary",),
        ),
    )(S.T, R)
    return jnp.concatenate([x, agg], axis=1)


# speedup vs baseline: 1.0021x; 1.0021x over previous
import jax
import jax.numpy as jnp
from jax.experimental import pallas as pl
from jax.experimental.pallas import tpu as pltpu

N_USERS = 4096
X_DIM = 64
K_SUBS = 20000
R_DIM = 3

BK = 1000
NK = K_SUBS // BK


def _body(st_ref, r_ref, o_ref, acc_ref):
    k = pl.program_id(0)

    @pl.when(k == 0)
    def _init():
        acc_ref[...] = jnp.zeros_like(acc_ref)

    r_blk = r_ref[pl.ds(pl.multiple_of(k * BK, 8), BK), :]
    acc_ref[...] += jax.lax.dot_general(
        st_ref[...],
        r_blk,
        (((0,), (0,)), ((), ())),
        preferred_element_type=jnp.float32,
    )

    @pl.when(k == NK - 1)
    def _fin():
        o_ref[...] = acc_ref[...]


def kernel(x, S, R):
    agg = pl.pallas_call(
        _body,
        grid=(NK,),
        in_specs=[
            pl.BlockSpec((BK, N_USERS), lambda k: (k, 0)),
            pl.BlockSpec((K_SUBS, R_DIM), lambda k: (0, 0)),
        ],
        out_specs=pl.BlockSpec((N_USERS, R_DIM), lambda k: (0, 0)),
        out_shape=jax.ShapeDtypeStruct((N_USERS, R_DIM), jnp.float32),
        scratch_shapes=[pltpu.VMEM((N_USERS, R_DIM), jnp.float32)],
        compiler_params=pltpu.CompilerParams(
            dimension_semantics=("arbitrary",),
        ),
    )(S.T, R)
    return jnp.concatenate([x, agg], axis=1)
